# baseline (speedup 1.0000x reference)
import functools

import jax
import jax.numpy as jnp
from jax import lax
from jax.experimental import pallas as pl
from jax.experimental.pallas import tpu as pltpu
from jax.experimental.pallas import tpu_sc as plsc

N = 10000
E = 320000
G = 64
IN_C = 128
HID = 256

NW = 32
NB = 320
NPAD = NW * NB
CAP = 12288
K = 128
RSTR = NB
TRASH = 16 * RSTR
EB = 3200
BR = 400

@functools.cache
def _get_mesh():
    return plsc.VectorSubcoreMesh(core_axis_name="c", subcore_axis_name="s")


@functools.cache
def _get_partition():
    return functools.partial(
        pl.kernel,
        out_type=(
            jax.ShapeDtypeStruct((NW, CAP), jnp.int32),
            jax.ShapeDtypeStruct((NW, CAP), jnp.int32),
            jax.ShapeDtypeStruct((NW, 16), jnp.int32),
            jax.ShapeDtypeStruct((NPAD,), jnp.float32),
        ),
        mesh=_get_mesh(),
        scratch_types=[
            pltpu.VMEM((2, EB), jnp.int32),
            pltpu.VMEM((2, EB), jnp.int32),
            pltpu.VMEM((CAP,), jnp.int32),
            pltpu.VMEM((CAP,), jnp.int32),
            pltpu.VMEM((NB,), jnp.float32),
            pltpu.VMEM((16,), jnp.int32),
            pltpu.SemaphoreType.DMA,
        ],
        compiler_params=pltpu.CompilerParams(needs_layout_passes=False),
    )(_partition_body)


_NBLK = E // EB
assert _NBLK * EB == E and _NBLK % 2 == 0


def _partition_body(src_hbm, dst_hbm, srcl_out, dstl_out, cnt_out, deg_out,
                    dst_v, src_v, csrc, cdstl, deg_v, cnt_v, bsem):
    wid = lax.axis_index("s") * 2 + lax.axis_index("c")
    lo = wid * NB

    zf = jnp.zeros((16,), jnp.float32)

    def _zero(i, carry):
        deg_v[pl.ds(i * 16, 16)] = zf
        return carry

    lax.fori_loop(0, NB // 16, _zero, 0)

    ones = jnp.ones((16,), jnp.float32)

    def _load(b, slot):
        pltpu.async_copy(dst_hbm.at[pl.ds(b * EB, EB)], dst_v.at[slot], bsem)
        pltpu.async_copy(src_hbm.at[pl.ds(b * EB, EB)], src_v.at[slot], bsem)

    def _wait_load(b, slot):
        pltpu.make_async_copy(
            dst_hbm.at[pl.ds(b * EB, EB)], dst_v.at[slot], bsem).wait()
        pltpu.make_async_copy(
            src_hbm.at[pl.ds(b * EB, EB)], src_v.at[slot], bsem).wait()

    _load(0, 0)

    _idx15 = jnp.full((16, 1), 15, jnp.int32)

    def _bcast_last(v):
        return lax.gather(
            v, _idx15,
            lax.GatherDimensionNumbers(offset_dims=(),
                                       collapsed_slice_dims=(0,),
                                       start_index_map=(0,)),
            (1,), mode=lax.GatherScatterMode.PROMISE_IN_BOUNDS)

    def _blockpair(i2, offv):
        for sl in range(2):
            b = i2 * 2 + sl

            @pl.when(b + 1 < _NBLK)
            def _():
                _load(b + 1, 1 - sl)

            _wait_load(b, sl)

            def _grp(i, offv):
                for u in range(2):
                    d = dst_v[sl, pl.ds(i * 32 + u * 16, 16)]
                    s = src_v[sl, pl.ds(i * 32 + u * 16, 16)]
                    m = (d >= lo) & (d < lo + NB)
                    dl = d - lo
                    cs = plsc.cumsum(m.astype(jnp.int32))
                    pos = offv + cs - 1
                    plsc.store_scatter(csrc, [pos], s, mask=m)
                    plsc.store_scatter(cdstl, [pos], dl, mask=m)
                    plsc.addupdate_scatter(deg_v, [dl], ones, mask=m)
                    offv = _bcast_last(pos) + 1
                return offv

            offv = lax.fori_loop(0, EB // 32, _grp, offv)
        return offv

    offv = lax.fori_loop(0, _NBLK // 2, _blockpair,
                         jnp.zeros((16,), jnp.int32))
    cnt = offv[0]

    pad_s = jnp.full((16,), lo, jnp.int32)
    pad_d = jnp.full((16,), NB, jnp.int32)

    def _pad(i, carry):
        csrc[pl.ds(cnt + i * 16, 16)] = pad_s
        cdstl[pl.ds(cnt + i * 16, 16)] = pad_d
        return carry

    lax.fori_loop(0, (K + 64) // 16, _pad, 0)

    pltpu.sync_copy(csrc, srcl_out.at[wid])
    pltpu.sync_copy(cdstl, dstl_out.at[wid])
    pltpu.sync_copy(deg_v, deg_out.at[pl.ds(lo, NB)])
    cnt_v[...] = offv
    pltpu.sync_copy(cnt_v, cnt_out.at[wid])


@functools.cache
def _make_agg(D):
    def _agg(h_hbm, srcl_hbm, dstl_hbm, cnt_hbm, out_hbm,
             rows, srcv, dstlv, idxv, cnt_v, spm,
             gsem0, gsem1, gsem2, ssem0, ssem1, ssem2):
        wid = lax.axis_index("s") * 2 + lax.axis_index("c")
        sid = lax.axis_index("s")
        lo = wid * NB
        base = sid * RSTR
        gsems = (gsem0, gsem1, gsem2)
        ssems = (ssem0, ssem1, ssem2)

        pltpu.sync_copy(srcl_hbm.at[wid], srcv)
        pltpu.sync_copy(dstl_hbm.at[wid], dstlv)
        pltpu.sync_copy(cnt_hbm.at[wid], cnt_v)

        zf = jnp.zeros((16,), jnp.float32)

        def _zero(r, carry):
            for c in range(D // 16):
                rows[0, r, pl.ds(c * 16, 16)] = zf
            return carry

        lax.fori_loop(0, K, _zero, 0)
        pltpu.sync_copy(rows.at[0], spm.at[pl.ds(base, K)])
        pltpu.sync_copy(rows.at[0], spm.at[pl.ds(base + K, K)])
        pltpu.sync_copy(rows.at[0].at[pl.ds(0, RSTR - 2 * K)],
                        spm.at[pl.ds(base + 2 * K, RSTR - 2 * K)])

        cnt = cnt_v[pl.ds(0, 16)][0]
        trips = (cnt + (K - 1)) >> 7

        def _issue(j, slot):
            for g in range(K // 16):
                dl = dstlv[pl.ds(j * K + g * 16, 16)]
                idxv[slot, pl.ds(g * 16, 16)] = jnp.where(
                    dl >= NB, TRASH, dl + base)
            pltpu.async_copy(h_hbm.at[srcv.at[pl.ds(j * K, K)]],
                             rows.at[slot], gsems[slot])

        def _wait_gather(j, slot):
            pltpu.make_async_copy(h_hbm.at[srcv.at[pl.ds(j * K, K)]],
                                  rows.at[slot], gsems[slot]).wait()

        def _scatter(slot):
            return pltpu.make_async_copy(
                rows.at[slot], spm.at[idxv.at[slot]], ssems[slot])

        for p in range(2):
            @pl.when(p < trips)
            def _(p=p):
                _issue(p, p)

        def _tri(j3, carry):
            for b in range(3):
                j = j3 * 3 + b

                @pl.when(j < trips)
                def _():
                    @pl.when(j + 2 < trips)
                    def _():
                        @pl.when(j >= 1)
                        def _():
                            _scatter((b + 2) % 3).wait()

                        _issue(j + 2, (b + 2) % 3)

                    _wait_gather(j, b)
                    pltpu.async_copy(rows.at[b], spm.at[idxv.at[b]],
                                     ssems[b], add=True)
            return carry

        lax.fori_loop(0, (trips + 2) // 3, _tri, 0)

        for b in range(3):
            @pl.when(b < trips)
            def _(b=b):
                _scatter(b).wait()

        pltpu.sync_copy(spm.at[pl.ds(base, NB)], out_hbm.at[pl.ds(lo, NB)])

    return functools.partial(
        pl.kernel,
        out_type=jax.ShapeDtypeStruct((NPAD, D), jnp.float32),
        mesh=_get_mesh(),
        scratch_types=[
            pltpu.VMEM((3, K, D), jnp.float32),
            pltpu.VMEM((CAP,), jnp.int32),
            pltpu.VMEM((CAP,), jnp.int32),
            pltpu.VMEM((3, K), jnp.int32),
            pltpu.VMEM((16,), jnp.int32),
            pltpu.VMEM_SHARED((16 * RSTR + 8, D), jnp.float32),
        ] + [pltpu.SemaphoreType.DMA] * 6,
        compiler_params=pltpu.CompilerParams(needs_layout_passes=False),
    )(_agg)


def _mm(a, b):
    return jnp.dot(a, b, preferred_element_type=jnp.float32)


def _tc_q_kernel(ha_ref, hb_ref, wrt_ref, wrb_ref, bl_ref, q_ref):
    q_ref[...] = (_mm(ha_ref[...], wrt_ref[...])
                  + _mm(hb_ref[...], wrb_ref[...]) + bl_ref[...])


def _tc_q1_kernel(x_ref, wr_ref, bl_ref, q_ref):
    q_ref[...] = _mm(x_ref[...], wr_ref[...]) + bl_ref[...]


def _tc_layer1_kernel(agg_ref, q_ref, degb_ref, wl_ref, outa_ref, outb_ref):
    invd = 1.0 / jnp.maximum(degb_ref[...], 1.0)
    p = _mm(agg_ref[...] * invd, wl_ref[...])
    h = jnp.maximum(p + q_ref[...], 0.0)
    outa_ref[...] = h[:, :128]
    outb_ref[...] = h[:, 128:]


def _tc_layer2_kernel(agga_ref, aggb_ref, q_ref, degb_ref,
                      wlt_ref, wlb_ref, outa_ref, outb_ref):
    invd = 1.0 / jnp.maximum(degb_ref[...], 1.0)
    p = _mm(agga_ref[...] * invd, wlt_ref[...]) + _mm(
        aggb_ref[...] * invd, wlb_ref[...])
    h = jnp.maximum(p + q_ref[...], 0.0)
    outa_ref[...] = h[:, :128]
    outb_ref[...] = h[:, 128:]


def _tc_layer3_kernel(agga_ref, aggb_ref, q_ref, degb_ref,
                      batch_ref, wlt_ref, wlb_ref, out_ref, cnt_scr):
    i = pl.program_id(0)

    @pl.when(i == 0)
    def _():
        out_ref[...] = jnp.zeros_like(out_ref)
        cnt_scr[...] = jnp.zeros_like(cnt_scr)

    invd = 1.0 / jnp.maximum(degb_ref[...], 1.0)
    p = _mm(agga_ref[...] * invd, wlt_ref[...]) + _mm(
        aggb_ref[...] * invd, wlb_ref[...])
    h3 = jnp.maximum(p + q_ref[...], 0.0)
    b = batch_ref[0, 0, :]
    seg = lax.broadcasted_iota(jnp.int32, (G, BR), 0)
    onehot = (seg == b[None, :]).astype(jnp.float32)
    out_ref[...] += _mm(onehot, h3)
    cnt_scr[...] += jnp.broadcast_to(
        jnp.sum(onehot, axis=1, keepdims=True), (G, 128))

    @pl.when(i == pl.num_programs(0) - 1)
    def _():
        out_ref[...] = out_ref[...] / jnp.maximum(cnt_scr[:, 0:1], 1.0)


_ROWB = lambda w: pl.BlockSpec((BR, w), lambda i: (i, 0))
_FULL = lambda r, c: pl.BlockSpec((r, c), lambda i: (0, 0))
_HN = jax.ShapeDtypeStruct((N, HID), jnp.float32)
_HHALF = [jax.ShapeDtypeStruct((N, 128), jnp.float32),
          jax.ShapeDtypeStruct((N, 128), jnp.float32)]


def _tc_q(ha, hb, wrt, wrb, blr):
    return pl.pallas_call(
        _tc_q_kernel, grid=(N // BR,),
        in_specs=[_ROWB(128), _ROWB(128),
                  _FULL(128, HID), _FULL(128, HID), _FULL(1, HID)],
        out_specs=_ROWB(HID), out_shape=_HN)(ha, hb, wrt, wrb, blr)


def _tc_q1(x, wr, blr):
    return pl.pallas_call(
        _tc_q1_kernel, grid=(N // BR,),
        in_specs=[_ROWB(128), _FULL(IN_C, HID), _FULL(1, HID)],
        out_specs=_ROWB(HID), out_shape=_HN)(x, wr, blr)


def _tc_layer1(agg, q, degb, wl):
    return pl.pallas_call(
        _tc_layer1_kernel, grid=(N // BR,),
        in_specs=[_ROWB(128), _ROWB(HID), _ROWB(128), _FULL(IN_C, HID)],
        out_specs=[_ROWB(128), _ROWB(128)],
        out_shape=_HHALF)(agg, q, degb, wl)


def _tc_layer2(agga, aggb, q, degb, wlt, wlb):
    return pl.pallas_call(
        _tc_layer2_kernel, grid=(N // BR,),
        in_specs=[_ROWB(128), _ROWB(128), _ROWB(HID), _ROWB(128),
                  _FULL(128, HID), _FULL(128, HID)],
        out_specs=[_ROWB(128), _ROWB(128)],
        out_shape=_HHALF)(agga, aggb, q, degb, wlt, wlb)


def _tc_layer3(agga, aggb, q, degb, batchb, wlt, wlb):
    return pl.pallas_call(
        _tc_layer3_kernel, grid=(N // BR,),
        in_specs=[_ROWB(128), _ROWB(128), _ROWB(HID), _ROWB(128),
                  pl.BlockSpec((1, 1, BR), lambda i: (i, 0, 0)),
                  _FULL(128, HID), _FULL(128, HID)],
        out_specs=pl.BlockSpec((G, HID), lambda i: (0, 0)),
        out_shape=jax.ShapeDtypeStruct((G, HID), jnp.float32),
        scratch_shapes=[pltpu.VMEM((G, 128), jnp.float32)],
    )(agga, aggb, q, degb, batchb, wlt, wlb)


def kernel(x, edge_index, batch, Wl1, bl1, Wr1, Wl2, bl2, Wr2, Wl3, bl3, Wr3):
    src = edge_index[0].astype(jnp.int32)
    dst = edge_index[1].astype(jnp.int32)
    srcl, dstl, cnt, deg = _get_partition()(src, dst)

    degb = jnp.broadcast_to(deg[:N, None], (N, 128))
    batchb = batch.astype(jnp.int32).reshape(N // BR, 1, BR)
    agg = _make_agg(128)

    q1 = _tc_q1(x, Wr1, bl1.reshape(1, HID))
    agg1 = agg(x, srcl, dstl, cnt)
    h1a, h1b = _tc_layer1(agg1, q1, degb, Wl1)
    q2 = _tc_q(h1a, h1b, Wr2[:128], Wr2[128:], bl2.reshape(1, HID))
    a2a = agg(h1a, srcl, dstl, cnt)
    a2b = agg(h1b, srcl, dstl, cnt)
    h2a, h2b = _tc_layer2(a2a, a2b, q2, degb, Wl2[:128], Wl2[128:])
    q3 = _tc_q(h2a, h2b, Wr3[:128], Wr3[128:], bl3.reshape(1, HID))
    a3a = agg(h2a, srcl, dstl, cnt)
    a3b = agg(h2b, srcl, dstl, cnt)
    return _tc_layer3(a3a, a3b, q3, degb, batchb, Wl3[:128], Wl3[128:])


# speedup vs baseline: 1.0669x; 1.0669x over previous
import functools

import jax
import jax.numpy as jnp
from jax import lax
from jax.experimental import pallas as pl
from jax.experimental.pallas import tpu as pltpu
from jax.experimental.pallas import tpu_sc as plsc

N = 10000
E = 320000
G = 64
IN_C = 128
HID = 256

NW = 32
NB = 320
NPAD = NW * NB
CAP = 12288
K = 128
RSTR = NB
TRASH = 16 * RSTR
EB = 3200
BR = 400

@functools.cache
def _get_mesh():
    return plsc.VectorSubcoreMesh(core_axis_name="c", subcore_axis_name="s")


@functools.cache
def _get_partition():
    return functools.partial(
        pl.kernel,
        out_type=(
            jax.ShapeDtypeStruct((NW, CAP), jnp.int32),
            jax.ShapeDtypeStruct((NW, CAP), jnp.int32),
            jax.ShapeDtypeStruct((NW, 16), jnp.int32),
            jax.ShapeDtypeStruct((NPAD,), jnp.float32),
        ),
        mesh=_get_mesh(),
        scratch_types=[
            pltpu.VMEM((2, EB), jnp.int32),
            pltpu.VMEM((2, EB), jnp.int32),
            pltpu.VMEM((CAP,), jnp.int32),
            pltpu.VMEM((CAP,), jnp.int32),
            pltpu.VMEM((NB,), jnp.float32),
            pltpu.VMEM((16,), jnp.int32),
            pltpu.SemaphoreType.DMA,
        ],
        compiler_params=pltpu.CompilerParams(needs_layout_passes=False),
    )(_partition_body)


_NBLK = E // EB
assert _NBLK * EB == E and _NBLK % 2 == 0


def _partition_body(src_hbm, dst_hbm, srcl_out, dstl_out, cnt_out, deg_out,
                    dst_v, src_v, csrc, cdstl, deg_v, cnt_v, bsem):
    wid = lax.axis_index("s") * 2 + lax.axis_index("c")
    lo = wid * NB

    zf = jnp.zeros((16,), jnp.float32)

    def _zero(i, carry):
        deg_v[pl.ds(i * 16, 16)] = zf
        return carry

    lax.fori_loop(0, NB // 16, _zero, 0)

    ones = jnp.ones((16,), jnp.float32)

    def _load(b, slot):
        pltpu.async_copy(dst_hbm.at[pl.ds(b * EB, EB)], dst_v.at[slot], bsem)
        pltpu.async_copy(src_hbm.at[pl.ds(b * EB, EB)], src_v.at[slot], bsem)

    def _wait_load(b, slot):
        pltpu.make_async_copy(
            dst_hbm.at[pl.ds(b * EB, EB)], dst_v.at[slot], bsem).wait()
        pltpu.make_async_copy(
            src_hbm.at[pl.ds(b * EB, EB)], src_v.at[slot], bsem).wait()

    _load(0, 0)

    _idx15 = jnp.full((16, 1), 15, jnp.int32)

    def _bcast_last(v):
        return lax.gather(
            v, _idx15,
            lax.GatherDimensionNumbers(offset_dims=(),
                                       collapsed_slice_dims=(0,),
                                       start_index_map=(0,)),
            (1,), mode=lax.GatherScatterMode.PROMISE_IN_BOUNDS)

    def _blockpair(i2, offv):
        for sl in range(2):
            b = i2 * 2 + sl

            @pl.when(b + 1 < _NBLK)
            def _():
                _load(b + 1, 1 - sl)

            _wait_load(b, sl)

            def _grp(i, offv):
                for u in range(2):
                    d = dst_v[sl, pl.ds(i * 32 + u * 16, 16)]
                    s = src_v[sl, pl.ds(i * 32 + u * 16, 16)]
                    m = (d >= lo) & (d < lo + NB)
                    dl = d - lo
                    cs = plsc.cumsum(m.astype(jnp.int32))
                    pos = offv + cs - 1
                    plsc.store_scatter(csrc, [pos], s, mask=m)
                    plsc.store_scatter(cdstl, [pos], dl, mask=m)
                    plsc.addupdate_scatter(deg_v, [dl], ones, mask=m)
                    offv = _bcast_last(pos) + 1
                return offv

            offv = lax.fori_loop(0, EB // 32, _grp, offv)
        return offv

    offv = lax.fori_loop(0, _NBLK // 2, _blockpair,
                         jnp.zeros((16,), jnp.int32))
    cnt = offv[0]

    pad_s = jnp.full((16,), lo, jnp.int32)
    pad_d = jnp.full((16,), NB, jnp.int32)

    def _pad(i, carry):
        csrc[pl.ds(cnt + i * 16, 16)] = pad_s
        cdstl[pl.ds(cnt + i * 16, 16)] = pad_d
        return carry

    lax.fori_loop(0, (K + 64) // 16, _pad, 0)

    pltpu.sync_copy(csrc, srcl_out.at[wid])
    pltpu.sync_copy(cdstl, dstl_out.at[wid])
    pltpu.sync_copy(deg_v, deg_out.at[pl.ds(lo, NB)])
    cnt_v[...] = offv
    pltpu.sync_copy(cnt_v, cnt_out.at[wid])


@functools.cache
def _make_agg(D):
    def _agg(h_hbm, srcl_hbm, dstl_hbm, cnt_hbm, out_hbm,
             rows, srcv, dstlv, idxv, cnt_v, spm,
             gsem0, gsem1, gsem2, ssem0, ssem1, ssem2):
        wid = lax.axis_index("s") * 2 + lax.axis_index("c")
        sid = lax.axis_index("s")
        lo = wid * NB
        base = sid * RSTR
        gsems = (gsem0, gsem1, gsem2)
        ssems = (ssem0, ssem1, ssem2)

        pltpu.sync_copy(srcl_hbm.at[wid], srcv)
        pltpu.sync_copy(dstl_hbm.at[wid], dstlv)
        pltpu.sync_copy(cnt_hbm.at[wid], cnt_v)

        zf = jnp.zeros((16,), jnp.float32)

        def _zero(r, carry):
            for c in range(D // 16):
                rows[0, r, pl.ds(c * 16, 16)] = zf
            return carry

        lax.fori_loop(0, K, _zero, 0)
        pltpu.sync_copy(rows.at[0], spm.at[pl.ds(base, K)])
        pltpu.sync_copy(rows.at[0], spm.at[pl.ds(base + K, K)])
        pltpu.sync_copy(rows.at[0].at[pl.ds(0, RSTR - 2 * K)],
                        spm.at[pl.ds(base + 2 * K, RSTR - 2 * K)])

        cnt = cnt_v[pl.ds(0, 16)][0]
        trips = (cnt + (K - 1)) >> 7

        def _issue(j, slot):
            for g in range(K // 16):
                dl = dstlv[pl.ds(j * K + g * 16, 16)]
                idxv[slot, pl.ds(g * 16, 16)] = jnp.where(
                    dl >= NB, TRASH, dl + base)
            pltpu.async_copy(h_hbm.at[srcv.at[pl.ds(j * K, K)]],
                             rows.at[slot], gsems[slot])

        def _wait_gather(j, slot):
            pltpu.make_async_copy(h_hbm.at[srcv.at[pl.ds(j * K, K)]],
                                  rows.at[slot], gsems[slot]).wait()

        def _scatter(slot):
            return pltpu.make_async_copy(
                rows.at[slot], spm.at[idxv.at[slot]], ssems[slot])

        for p in range(2):
            @pl.when(p < trips)
            def _(p=p):
                _issue(p, p)

        def _tri(j3, carry):
            for b in range(3):
                j = j3 * 3 + b

                @pl.when(j < trips)
                def _():
                    @pl.when(j + 2 < trips)
                    def _():
                        @pl.when(j >= 1)
                        def _():
                            _scatter((b + 2) % 3).wait()

                        _issue(j + 2, (b + 2) % 3)

                    _wait_gather(j, b)
                    pltpu.async_copy(rows.at[b], spm.at[idxv.at[b]],
                                     ssems[b], add=True)
            return carry

        lax.fori_loop(0, (trips + 2) // 3, _tri, 0)

        for b in range(3):
            @pl.when(b < trips)
            def _(b=b):
                _scatter(b).wait()

        pltpu.sync_copy(spm.at[pl.ds(base, NB)], out_hbm.at[pl.ds(lo, NB)])

    return functools.partial(
        pl.kernel,
        out_type=jax.ShapeDtypeStruct((NPAD, D), jnp.float32),
        mesh=_get_mesh(),
        scratch_types=[
            pltpu.VMEM((3, K, D), jnp.float32),
            pltpu.VMEM((CAP,), jnp.int32),
            pltpu.VMEM((CAP,), jnp.int32),
            pltpu.VMEM((3, K), jnp.int32),
            pltpu.VMEM((16,), jnp.int32),
            pltpu.VMEM_SHARED((16 * RSTR + 8, D), jnp.float32),
        ] + [pltpu.SemaphoreType.DMA] * 6,
        compiler_params=pltpu.CompilerParams(needs_layout_passes=False),
    )(_agg)


def _mm(a, b):
    return jnp.dot(a, b, preferred_element_type=jnp.float32)


def _tc_layer1_kernel(agg_ref, x_ref, degb_ref, wl_ref, wr_ref, bl_ref,
                      outa_ref, outb_ref):
    invd = 1.0 / jnp.maximum(degb_ref[...], 1.0)
    p = _mm(agg_ref[...] * invd, wl_ref[...])
    q = _mm(x_ref[...], wr_ref[...])
    h = jnp.maximum(p + q + bl_ref[...], 0.0)
    outa_ref[...] = h[:, :128]
    outb_ref[...] = h[:, 128:]


def _tc_layer2_kernel(agga_ref, aggb_ref, ha_ref, hb_ref, degb_ref,
                      wlt_ref, wlb_ref, wrt_ref, wrb_ref, bl_ref,
                      outa_ref, outb_ref):
    invd = 1.0 / jnp.maximum(degb_ref[...], 1.0)
    p = _mm(agga_ref[...] * invd, wlt_ref[...]) + _mm(
        aggb_ref[...] * invd, wlb_ref[...])
    q = _mm(ha_ref[...], wrt_ref[...]) + _mm(hb_ref[...], wrb_ref[...])
    h = jnp.maximum(p + q + bl_ref[...], 0.0)
    outa_ref[...] = h[:, :128]
    outb_ref[...] = h[:, 128:]


def _tc_layer3_kernel(agga_ref, aggb_ref, ha_ref, hb_ref, degb_ref,
                      batch_ref, wlt_ref, wlb_ref, wrt_ref, wrb_ref, bl_ref,
                      out_ref, cnt_scr):
    i = pl.program_id(0)

    @pl.when(i == 0)
    def _():
        out_ref[...] = jnp.zeros_like(out_ref)
        cnt_scr[...] = jnp.zeros_like(cnt_scr)

    invd = 1.0 / jnp.maximum(degb_ref[...], 1.0)
    p = _mm(agga_ref[...] * invd, wlt_ref[...]) + _mm(
        aggb_ref[...] * invd, wlb_ref[...])
    q = _mm(ha_ref[...], wrt_ref[...]) + _mm(hb_ref[...], wrb_ref[...])
    h3 = jnp.maximum(p + q + bl_ref[...], 0.0)
    b = batch_ref[0, 0, :]
    seg = lax.broadcasted_iota(jnp.int32, (G, BR), 0)
    onehot = (seg == b[None, :]).astype(jnp.float32)
    out_ref[...] += _mm(onehot, h3)
    cnt_scr[...] += jnp.broadcast_to(
        jnp.sum(onehot, axis=1, keepdims=True), (G, 128))

    @pl.when(i == pl.num_programs(0) - 1)
    def _():
        out_ref[...] = out_ref[...] / jnp.maximum(cnt_scr[:, 0:1], 1.0)


_ROWB = lambda w: pl.BlockSpec((BR, w), lambda i: (i, 0))
_FULL = lambda r, c: pl.BlockSpec((r, c), lambda i: (0, 0))
_HHALF = [jax.ShapeDtypeStruct((N, 128), jnp.float32),
          jax.ShapeDtypeStruct((N, 128), jnp.float32)]


def _tc_layer1(agg, x, degb, wl, wr, blr):
    return pl.pallas_call(
        _tc_layer1_kernel,
        grid=(N // BR,),
        in_specs=[_ROWB(128), _ROWB(128), _ROWB(128),
                  _FULL(IN_C, HID), _FULL(IN_C, HID), _FULL(1, HID)],
        out_specs=[_ROWB(128), _ROWB(128)],
        out_shape=_HHALF)(agg, x, degb, wl, wr, blr)


def _tc_layer2(agga, aggb, ha, hb, degb, wlt, wlb, wrt, wrb, blr):
    return pl.pallas_call(
        _tc_layer2_kernel,
        grid=(N // BR,),
        in_specs=[_ROWB(128)] * 5 + [_FULL(128, HID)] * 4 + [_FULL(1, HID)],
        out_specs=[_ROWB(128), _ROWB(128)],
        out_shape=_HHALF)(agga, aggb, ha, hb, degb, wlt, wlb, wrt, wrb, blr)


def _tc_layer3(agga, aggb, ha, hb, degb, batchb, wlt, wlb, wrt, wrb, blr):
    return pl.pallas_call(
        _tc_layer3_kernel,
        grid=(N // BR,),
        in_specs=[_ROWB(128)] * 5
        + [pl.BlockSpec((1, 1, BR), lambda i: (i, 0, 0))]
        + [_FULL(128, HID)] * 4 + [_FULL(1, HID)],
        out_specs=pl.BlockSpec((G, HID), lambda i: (0, 0)),
        out_shape=jax.ShapeDtypeStruct((G, HID), jnp.float32),
        scratch_shapes=[pltpu.VMEM((G, 128), jnp.float32)],
    )(agga, aggb, ha, hb, degb, batchb, wlt, wlb, wrt, wrb, blr)


def kernel(x, edge_index, batch, Wl1, bl1, Wr1, Wl2, bl2, Wr2, Wl3, bl3, Wr3):
    src = edge_index[0].astype(jnp.int32)
    dst = edge_index[1].astype(jnp.int32)
    srcl, dstl, cnt, deg = _get_partition()(src, dst)

    degb = jnp.broadcast_to(deg[:N, None], (N, 128))
    batchb = batch.astype(jnp.int32).reshape(N // BR, 1, BR)
    agg = _make_agg(128)

    agg1 = agg(x, srcl, dstl, cnt)
    h1a, h1b = _tc_layer1(agg1, x, degb, Wl1, Wr1, bl1.reshape(1, HID))
    a2a = agg(h1a, srcl, dstl, cnt)
    a2b = agg(h1b, srcl, dstl, cnt)
    h2a, h2b = _tc_layer2(a2a, a2b, h1a, h1b, degb,
                          Wl2[:128], Wl2[128:], Wr2[:128], Wr2[128:],
                          bl2.reshape(1, HID))
    a3a = agg(h2a, srcl, dstl, cnt)
    a3b = agg(h2b, srcl, dstl, cnt)
    return _tc_layer3(a3a, a3b, h2a, h2b, degb, batchb,
                      Wl3[:128], Wl3[128:], Wr3[:128], Wr3[128:],
                      bl3.reshape(1, HID))


# speedup vs baseline: 1.0816x; 1.0138x over previous
import functools

import jax
import jax.numpy as jnp
from jax import lax
from jax.experimental import pallas as pl
from jax.experimental.pallas import tpu as pltpu
from jax.experimental.pallas import tpu_sc as plsc

N = 10000
E = 320000
G = 64
IN_C = 128
HID = 256

NW = 32
NB = 320
NPAD = NW * NB
CAP = 12288
K = 128
RSTR = NB
TRASH = 16 * RSTR
EB = 3200
BR = 400

@functools.cache
def _get_mesh():
    return plsc.VectorSubcoreMesh(core_axis_name="c", subcore_axis_name="s")


@functools.cache
def _get_partition():
    return functools.partial(
        pl.kernel,
        out_type=(
            jax.ShapeDtypeStruct((NW, CAP), jnp.int32),
            jax.ShapeDtypeStruct((NW, CAP), jnp.int32),
            jax.ShapeDtypeStruct((NW, 16), jnp.int32),
            jax.ShapeDtypeStruct((NPAD,), jnp.float32),
        ),
        mesh=_get_mesh(),
        scratch_types=[
            pltpu.VMEM((2, EB), jnp.int32),
            pltpu.VMEM((2, EB), jnp.int32),
            pltpu.VMEM((CAP,), jnp.int32),
            pltpu.VMEM((CAP,), jnp.int32),
            pltpu.VMEM((NB,), jnp.float32),
            pltpu.VMEM((16,), jnp.int32),
            pltpu.SemaphoreType.DMA,
        ],
        compiler_params=pltpu.CompilerParams(needs_layout_passes=False),
    )(_partition_body)


_NBLK = E // EB
assert _NBLK * EB == E and _NBLK % 2 == 0


def _partition_body(src_hbm, dst_hbm, srcl_out, dstl_out, cnt_out, deg_out,
                    dst_v, src_v, csrc, cdstl, deg_v, cnt_v, bsem):
    wid = lax.axis_index("s") * 2 + lax.axis_index("c")
    lo = wid * NB

    zf = jnp.zeros((16,), jnp.float32)

    def _zero(i, carry):
        deg_v[pl.ds(i * 16, 16)] = zf
        return carry

    lax.fori_loop(0, NB // 16, _zero, 0)

    ones = jnp.ones((16,), jnp.float32)

    def _load(b, slot):
        pltpu.async_copy(dst_hbm.at[pl.ds(b * EB, EB)], dst_v.at[slot], bsem)
        pltpu.async_copy(src_hbm.at[pl.ds(b * EB, EB)], src_v.at[slot], bsem)

    def _wait_load(b, slot):
        pltpu.make_async_copy(
            dst_hbm.at[pl.ds(b * EB, EB)], dst_v.at[slot], bsem).wait()
        pltpu.make_async_copy(
            src_hbm.at[pl.ds(b * EB, EB)], src_v.at[slot], bsem).wait()

    _load(0, 0)

    _idx15 = jnp.full((16, 1), 15, jnp.int32)

    def _bcast_last(v):
        return lax.gather(
            v, _idx15,
            lax.GatherDimensionNumbers(offset_dims=(),
                                       collapsed_slice_dims=(0,),
                                       start_index_map=(0,)),
            (1,), mode=lax.GatherScatterMode.PROMISE_IN_BOUNDS)

    def _blockpair(i2, offv):
        for sl in range(2):
            b = i2 * 2 + sl

            @pl.when(b + 1 < _NBLK)
            def _():
                _load(b + 1, 1 - sl)

            _wait_load(b, sl)

            def _grp(i, offv):
                d = dst_v[sl, pl.ds(i * 16, 16)]
                s = src_v[sl, pl.ds(i * 16, 16)]
                m = (d >= lo) & (d < lo + NB)
                dl = d - lo
                cs = plsc.cumsum(m.astype(jnp.int32))
                pos = offv + cs - 1
                plsc.store_scatter(csrc, [pos], s, mask=m)
                plsc.store_scatter(cdstl, [pos], dl, mask=m)
                plsc.addupdate_scatter(deg_v, [dl], ones, mask=m)
                return _bcast_last(pos) + 1

            offv = lax.fori_loop(0, EB // 16, _grp, offv)
        return offv

    offv = lax.fori_loop(0, _NBLK // 2, _blockpair,
                         jnp.zeros((16,), jnp.int32))
    cnt = offv[0]

    pad_s = jnp.full((16,), lo, jnp.int32)
    pad_d = jnp.full((16,), NB, jnp.int32)

    def _pad(i, carry):
        csrc[pl.ds(cnt + i * 16, 16)] = pad_s
        cdstl[pl.ds(cnt + i * 16, 16)] = pad_d
        return carry

    lax.fori_loop(0, (K + 64) // 16, _pad, 0)

    pltpu.sync_copy(csrc, srcl_out.at[wid])
    pltpu.sync_copy(cdstl, dstl_out.at[wid])
    pltpu.sync_copy(deg_v, deg_out.at[pl.ds(lo, NB)])
    cnt_v[...] = offv
    pltpu.sync_copy(cnt_v, cnt_out.at[wid])


@functools.cache
def _make_agg(D):
    def _agg(h_hbm, srcl_hbm, dstl_hbm, cnt_hbm, out_hbm,
             rows, srcv, dstlv, idxv, cnt_v, spm,
             gsem0, gsem1, gsem2, ssem0, ssem1, ssem2):
        wid = lax.axis_index("s") * 2 + lax.axis_index("c")
        sid = lax.axis_index("s")
        lo = wid * NB
        base = sid * RSTR
        gsems = (gsem0, gsem1, gsem2)
        ssems = (ssem0, ssem1, ssem2)

        pltpu.sync_copy(srcl_hbm.at[wid], srcv)
        pltpu.sync_copy(dstl_hbm.at[wid], dstlv)
        pltpu.sync_copy(cnt_hbm.at[wid], cnt_v)

        zf = jnp.zeros((16,), jnp.float32)

        def _zero(r, carry):
            for c in range(D // 16):
                rows[0, r, pl.ds(c * 16, 16)] = zf
            return carry

        lax.fori_loop(0, K, _zero, 0)
        pltpu.sync_copy(rows.at[0], spm.at[pl.ds(base, K)])
        pltpu.sync_copy(rows.at[0], spm.at[pl.ds(base + K, K)])
        pltpu.sync_copy(rows.at[0].at[pl.ds(0, RSTR - 2 * K)],
                        spm.at[pl.ds(base + 2 * K, RSTR - 2 * K)])

        cnt = cnt_v[pl.ds(0, 16)][0]
        trips = (cnt + (K - 1)) >> 7

        def _issue(j, slot):
            for g in range(K // 16):
                dl = dstlv[pl.ds(j * K + g * 16, 16)]
                idxv[slot, pl.ds(g * 16, 16)] = jnp.where(
                    dl >= NB, TRASH, dl + base)
            pltpu.async_copy(h_hbm.at[srcv.at[pl.ds(j * K, K)]],
                             rows.at[slot], gsems[slot])

        def _wait_gather(j, slot):
            pltpu.make_async_copy(h_hbm.at[srcv.at[pl.ds(j * K, K)]],
                                  rows.at[slot], gsems[slot]).wait()

        def _scatter(slot):
            return pltpu.make_async_copy(
                rows.at[slot], spm.at[idxv.at[slot]], ssems[slot])

        for p in range(2):
            @pl.when(p < trips)
            def _(p=p):
                _issue(p, p)

        def _tri(j3, carry):
            for b in range(3):
                j = j3 * 3 + b

                @pl.when(j < trips)
                def _():
                    @pl.when(j + 2 < trips)
                    def _():
                        @pl.when(j >= 1)
                        def _():
                            _scatter((b + 2) % 3).wait()

                        _issue(j + 2, (b + 2) % 3)

                    _wait_gather(j, b)
                    pltpu.async_copy(rows.at[b], spm.at[idxv.at[b]],
                                     ssems[b], add=True)
            return carry

        lax.fori_loop(0, (trips + 2) // 3, _tri, 0)

        for b in range(3):
            @pl.when(b < trips)
            def _(b=b):
                _scatter(b).wait()

        pltpu.sync_copy(spm.at[pl.ds(base, NB)], out_hbm.at[pl.ds(lo, NB)])

    return functools.partial(
        pl.kernel,
        out_type=jax.ShapeDtypeStruct((NPAD, D), jnp.float32),
        mesh=_get_mesh(),
        scratch_types=[
            pltpu.VMEM((3, K, D), jnp.float32),
            pltpu.VMEM((CAP,), jnp.int32),
            pltpu.VMEM((CAP,), jnp.int32),
            pltpu.VMEM((3, K), jnp.int32),
            pltpu.VMEM((16,), jnp.int32),
            pltpu.VMEM_SHARED((16 * RSTR + 8, D), jnp.float32),
        ] + [pltpu.SemaphoreType.DMA] * 6,
        compiler_params=pltpu.CompilerParams(needs_layout_passes=False),
    )(_agg)


def _mm(a, b):
    return jnp.dot(a, b, preferred_element_type=jnp.float32)


def _tc_layer1_kernel(agg_ref, x_ref, degb_ref, wl_ref, wr_ref, bl_ref,
                      outa_ref, outb_ref):
    invd = 1.0 / jnp.maximum(degb_ref[...], 1.0)
    p = _mm(agg_ref[...] * invd, wl_ref[...])
    q = _mm(x_ref[...], wr_ref[...])
    h = jnp.maximum(p + q + bl_ref[...], 0.0)
    outa_ref[...] = h[:, :128]
    outb_ref[...] = h[:, 128:]


def _tc_layer2_kernel(agga_ref, aggb_ref, ha_ref, hb_ref, degb_ref,
                      wlt_ref, wlb_ref, wrt_ref, wrb_ref, bl_ref,
                      outa_ref, outb_ref):
    invd = 1.0 / jnp.maximum(degb_ref[...], 1.0)
    p = _mm(agga_ref[...] * invd, wlt_ref[...]) + _mm(
        aggb_ref[...] * invd, wlb_ref[...])
    q = _mm(ha_ref[...], wrt_ref[...]) + _mm(hb_ref[...], wrb_ref[...])
    h = jnp.maximum(p + q + bl_ref[...], 0.0)
    outa_ref[...] = h[:, :128]
    outb_ref[...] = h[:, 128:]


def _tc_layer3_kernel(agga_ref, aggb_ref, ha_ref, hb_ref, degb_ref,
                      batch_ref, wlt_ref, wlb_ref, wrt_ref, wrb_ref, bl_ref,
                      out_ref, cnt_scr):
    i = pl.program_id(0)

    @pl.when(i == 0)
    def _():
        out_ref[...] = jnp.zeros_like(out_ref)
        cnt_scr[...] = jnp.zeros_like(cnt_scr)

    invd = 1.0 / jnp.maximum(degb_ref[...], 1.0)
    p = _mm(agga_ref[...] * invd, wlt_ref[...]) + _mm(
        aggb_ref[...] * invd, wlb_ref[...])
    q = _mm(ha_ref[...], wrt_ref[...]) + _mm(hb_ref[...], wrb_ref[...])
    h3 = jnp.maximum(p + q + bl_ref[...], 0.0)
    b = batch_ref[0, 0, :]
    seg = lax.broadcasted_iota(jnp.int32, (G, BR), 0)
    onehot = (seg == b[None, :]).astype(jnp.float32)
    out_ref[...] += _mm(onehot, h3)
    cnt_scr[...] += jnp.broadcast_to(
        jnp.sum(onehot, axis=1, keepdims=True), (G, 128))

    @pl.when(i == pl.num_programs(0) - 1)
    def _():
        out_ref[...] = out_ref[...] / jnp.maximum(cnt_scr[:, 0:1], 1.0)


_ROWB = lambda w: pl.BlockSpec((BR, w), lambda i: (i, 0))
_FULL = lambda r, c: pl.BlockSpec((r, c), lambda i: (0, 0))
_HHALF = [jax.ShapeDtypeStruct((N, 128), jnp.float32),
          jax.ShapeDtypeStruct((N, 128), jnp.float32)]


def _tc_layer1(agg, x, degb, wl, wr, blr):
    return pl.pallas_call(
        _tc_layer1_kernel,
        grid=(N // BR,),
        in_specs=[_ROWB(128), _ROWB(128), _ROWB(128),
                  _FULL(IN_C, HID), _FULL(IN_C, HID), _FULL(1, HID)],
        out_specs=[_ROWB(128), _ROWB(128)],
        out_shape=_HHALF)(agg, x, degb, wl, wr, blr)


def _tc_layer2(agga, aggb, ha, hb, degb, wlt, wlb, wrt, wrb, blr):
    return pl.pallas_call(
        _tc_layer2_kernel,
        grid=(N // BR,),
        in_specs=[_ROWB(128)] * 5 + [_FULL(128, HID)] * 4 + [_FULL(1, HID)],
        out_specs=[_ROWB(128), _ROWB(128)],
        out_shape=_HHALF)(agga, aggb, ha, hb, degb, wlt, wlb, wrt, wrb, blr)


def _tc_layer3(agga, aggb, ha, hb, degb, batchb, wlt, wlb, wrt, wrb, blr):
    return pl.pallas_call(
        _tc_layer3_kernel,
        grid=(N // BR,),
        in_specs=[_ROWB(128)] * 5
        + [pl.BlockSpec((1, 1, BR), lambda i: (i, 0, 0))]
        + [_FULL(128, HID)] * 4 + [_FULL(1, HID)],
        out_specs=pl.BlockSpec((G, HID), lambda i: (0, 0)),
        out_shape=jax.ShapeDtypeStruct((G, HID), jnp.float32),
        scratch_shapes=[pltpu.VMEM((G, 128), jnp.float32)],
    )(agga, aggb, ha, hb, degb, batchb, wlt, wlb, wrt, wrb, blr)


def kernel(x, edge_index, batch, Wl1, bl1, Wr1, Wl2, bl2, Wr2, Wl3, bl3, Wr3):
    src = edge_index[0].astype(jnp.int32)
    dst = edge_index[1].astype(jnp.int32)
    srcl, dstl, cnt, deg = _get_partition()(src, dst)

    degb = jnp.broadcast_to(deg[:N, None], (N, 128))
    batchb = batch.astype(jnp.int32).reshape(N // BR, 1, BR)
    agg = _make_agg(128)

    agg1 = agg(x, srcl, dstl, cnt)
    h1a, h1b = _tc_layer1(agg1, x, degb, Wl1, Wr1, bl1.reshape(1, HID))
    a2a = agg(h1a, srcl, dstl, cnt)
    a2b = agg(h1b, srcl, dstl, cnt)
    h2a, h2b = _tc_layer2(a2a, a2b, h1a, h1b, degb,
                          Wl2[:128], Wl2[128:], Wr2[:128], Wr2[128:],
                          bl2.reshape(1, HID))
    a3a = agg(h2a, srcl, dstl, cnt)
    a3b = agg(h2b, srcl, dstl, cnt)
    return _tc_layer3(a3a, a3b, h2a, h2b, degb, batchb,
                      Wl3[:128], Wl3[128:], Wr3[:128], Wr3[128:],
                      bl3.reshape(1, HID))
